# Initial kernel scaffold; baseline (speedup 1.0000x reference)
import functools

import jax
import jax.numpy as jnp
from jax import lax
from jax.experimental import pallas as pl
from jax.experimental.pallas import tpu as pltpu
from jax.experimental.pallas import tpu_sc as plsc

_DIM = 3
_POS_HID = 32
_ATTN_HID = 12
_B = 4
_N = 512
_L = 16
_NC = 2
_NS = 16
_TPB = (_NC * _NS) // _B
_PPT = _N // _TPB

_OQKV = 0
_OW1 = 27
_OB1 = 123
_OW2 = 155
_OB2 = 251
_OA1 = 254
_OAB1 = 290
_OA2 = 302
_OAB2 = 338
_PER_LAYER = 341
_OLW = 3 * _PER_LAYER
_OLB = _OLW + 6
_NW = _OLB + 2


def _pack_weights(params):
  segs = []
  for ln in ('l1', 'l2', 'l3'):
    p = params[ln]
    segs += [p['qkv'].reshape(-1), p['pos_w1'].reshape(-1), p['pos_b1'],
             p['pos_w2'].reshape(-1), p['pos_b2'], p['attn_w1'].reshape(-1),
             p['attn_b1'], p['attn_w2'].reshape(-1), p['attn_b2']]
  segs += [params['lin_w'].reshape(-1), params['lin_b']]
  w = jnp.concatenate(segs).astype(jnp.float32)
  return jnp.repeat(w, _L)


def _sc_body(pos_hbm, x_hbm, w_hbm, out_hbm,
             posb, xb, qkvb, idxb, rpb, outb, xnb, fb, wv, xsh):
  c = lax.axis_index('c')
  s = lax.axis_index('s')
  bl = s // _TPB
  b = c * 2 + bl
  ch = s % _TPB
  base = ch * _PPT

  pltpu.sync_copy(pos_hbm.at[b], posb)
  pltpu.sync_copy(x_hbm.at[b], xb)
  pltpu.sync_copy(w_hbm, wv)

  def wsp(j):
    return wv[pl.ds(_L * j, _L)]

  iota = lax.iota(jnp.int32, _L)

  def row(r):
    return jnp.full((_L,), r, jnp.int32)

  def topk_body(i, _):
    gi = base + i
    gidx = jnp.full((_L,), gi, jnp.int32)
    px = plsc.load_gather(posb, [row(0), gidx])
    py = plsc.load_gather(posb, [row(1), gidx])
    pz = plsc.load_gather(posb, [row(2), gidx])

    def chunk_body(j, carry):
      bk, bv = carry
      off = j * _L
      dx = px - posb[0, pl.ds(off, _L)]
      dy = py - posb[1, pl.ds(off, _L)]
      dz = pz - posb[2, pl.ds(off, _L)]
      d2 = dx * dx + dy * dy + dz * dz
      ck, cv = plsc.sort_key_val(d2, iota + off)
      ckr = lax.rev(ck, (0,))
      cvr = lax.rev(cv, (0,))
      keep = bk <= ckr
      mk = jnp.where(keep, bk, ckr)
      mv = jnp.where(keep, bv, cvr)
      return plsc.sort_key_val(mk, mv)

    bk0 = jnp.full((_L,), jnp.inf, jnp.float32)
    bv0 = jnp.zeros((_L,), jnp.int32)
    _, bv = lax.fori_loop(0, _N // _L, chunk_body, (bk0, bv0))
    idxb[i, :] = bv
    gx = plsc.load_gather(posb, [row(0), bv])
    gy = plsc.load_gather(posb, [row(1), bv])
    gz = plsc.load_gather(posb, [row(2), bv])
    rpb[0, i, :] = px - gx
    rpb[1, i, :] = py - gy
    rpb[2, i, :] = pz - gz
    return 0

  lax.fori_loop(0, _PPT, topk_body, 0)

  for l in range(3):
    lb = l * _PER_LAYER

    def qkv_body(j, _, lb=lb):
      off = j * _L
      x0 = xb[0, pl.ds(off, _L)]
      x1 = xb[1, pl.ds(off, _L)]
      x2 = xb[2, pl.ds(off, _L)]
      for u in range(9):
        qkvb[u, pl.ds(off, _L)] = (x0 * wsp(lb + _OQKV + u) +
                                   x1 * wsp(lb + _OQKV + 9 + u) +
                                   x2 * wsp(lb + _OQKV + 18 + u))
      return 0

    lax.fori_loop(0, _N // _L, qkv_body, 0)

    def pt_body(i, _, lb=lb):
      gi = base + i
      gidx = jnp.full((_L,), gi, jnp.int32)
      nb = idxb[i, :]
      qx = plsc.load_gather(qkvb, [row(0), gidx])
      qy = plsc.load_gather(qkvb, [row(1), gidx])
      qz = plsc.load_gather(qkvb, [row(2), gidx])
      kx = plsc.load_gather(qkvb, [row(3), nb])
      ky = plsc.load_gather(qkvb, [row(4), nb])
      kz = plsc.load_gather(qkvb, [row(5), nb])
      vx = plsc.load_gather(qkvb, [row(6), nb])
      vy = plsc.load_gather(qkvb, [row(7), nb])
      vz = plsc.load_gather(qkvb, [row(8), nb])
      rx = rpb[0, i, :]
      ry = rpb[1, i, :]
      rz = rpb[2, i, :]
      pex = wsp(lb + _OB2 + 0)
      pey = wsp(lb + _OB2 + 1)
      pez = wsp(lb + _OB2 + 2)
      for u in range(_POS_HID):
        h = (rx * wsp(lb + _OW1 + u) + ry * wsp(lb + _OW1 + 32 + u) +
             rz * wsp(lb + _OW1 + 64 + u) + wsp(lb + _OB1 + u))
        h = jnp.maximum(h, 0.0)
        pex = pex + h * wsp(lb + _OW2 + u * 3)
        pey = pey + h * wsp(lb + _OW2 + u * 3 + 1)
        pez = pez + h * wsp(lb + _OW2 + u * 3 + 2)
      sx = qx - kx + pex
      sy = qy - ky + pey
      sz = qz - kz + pez
      ox = wsp(lb + _OAB2 + 0)
      oy = wsp(lb + _OAB2 + 1)
      oz = wsp(lb + _OAB2 + 2)
      for u in range(_ATTN_HID):
        g = (sx * wsp(lb + _OA1 + u) + sy * wsp(lb + _OA1 + 12 + u) +
             sz * wsp(lb + _OA1 + 24 + u) + wsp(lb + _OAB1 + u))
        g = jnp.maximum(g, 0.0)
        ox = ox + g * wsp(lb + _OA2 + u * 3)
        oy = oy + g * wsp(lb + _OA2 + u * 3 + 1)
        oz = oz + g * wsp(lb + _OA2 + u * 3 + 2)

      def chan(sim, vv, pe):
        m = jnp.max(sim)
        e = jnp.exp(sim - m)
        a = e / jnp.sum(e)
        return jnp.sum(a * (vv + pe))

      o0 = chan(ox, vx, pex)
      o1 = chan(oy, vy, pey)
      o2 = chan(oz, vz, pez)
      vout = jnp.where(iota == 0, o0, jnp.where(iota == 1, o1, o2))
      plsc.store_scatter(outb,
                         [jnp.minimum(iota, 2), jnp.full((_L,), i, jnp.int32)],
                         vout, mask=iota < 3)
      return 0

    lax.fori_loop(0, _PPT, pt_body, 0)

    if l < 2:
      for t in range(_PPT // _L):
        for cc in range(3):
          z = outb[cc, pl.ds(t * _L, _L)]
          xnb[cc, pl.ds(t * _L, _L)] = 1.0 / (1.0 + jnp.exp(-z))
      pltpu.sync_copy(xnb, xsh.at[bl, :, pl.ds(base, _PPT)])
      plsc.subcore_barrier()
      pltpu.sync_copy(xsh.at[bl], xb)
      plsc.subcore_barrier()
    else:
      for t in range(_PPT // _L):
        xs = []
        for cc in range(3):
          z = outb[cc, pl.ds(t * _L, _L)]
          xs.append(1.0 / (1.0 + jnp.exp(-z)))
        u0 = (xs[0] * wsp(_OLW + 0) + xs[1] * wsp(_OLW + 2) +
              xs[2] * wsp(_OLW + 4) + wsp(_OLB + 0))
        u1 = (xs[0] * wsp(_OLW + 1) + xs[1] * wsp(_OLW + 3) +
              xs[2] * wsp(_OLW + 5) + wsp(_OLB + 1))
        m = jnp.maximum(u0, u1)
        e0 = jnp.exp(u0 - m)
        e1 = jnp.exp(u1 - m)
        tot = e0 + e1
        fb[0, pl.ds(t * _L, _L)] = e0 / tot
        fb[1, pl.ds(t * _L, _L)] = e1 / tot
      pltpu.sync_copy(fb, out_hbm.at[b, :, pl.ds(base, _PPT)])


@jax.jit
def _sc_call(pos_t, x_t, wflat):
  mesh = plsc.VectorSubcoreMesh(core_axis_name='c', subcore_axis_name='s',
                                num_cores=_NC, num_subcores=_NS)
  return pl.kernel(
      _sc_body,
      out_type=jax.ShapeDtypeStruct((_B, 2, _N), jnp.float32),
      mesh=mesh,
      scratch_types=[
          pltpu.VMEM((3, _N), jnp.float32),
          pltpu.VMEM((3, _N), jnp.float32),
          pltpu.VMEM((9, _N), jnp.float32),
          pltpu.VMEM((_PPT, _L), jnp.int32),
          pltpu.VMEM((3, _PPT, _L), jnp.float32),
          pltpu.VMEM((3, _PPT), jnp.float32),
          pltpu.VMEM((3, _PPT), jnp.float32),
          pltpu.VMEM((2, _PPT), jnp.float32),
          pltpu.VMEM((_NW * _L,), jnp.float32),
          pltpu.VMEM_SHARED((2, 3, _N), jnp.float32),
      ],
      name='pt_knn_sc',
  )(pos_t, x_t, wflat)


def kernel(feats, pos, mask, params):
  del mask
  pos_t = jnp.transpose(pos, (0, 2, 1)).astype(jnp.float32)
  x_t = jnp.transpose(feats, (0, 2, 1)).astype(jnp.float32)
  wflat = _pack_weights(params)
  out = _sc_call(pos_t, x_t, wflat)
  return jnp.transpose(out, (0, 2, 1))


# speedup vs baseline: 2.9180x; 2.9180x over previous
import jax
import jax.numpy as jnp
from jax import lax
from jax.experimental import pallas as pl
from jax.experimental.pallas import tpu as pltpu
from jax.experimental.pallas import tpu_sc as plsc

_POS_HID = 32
_ATTN_HID = 12
_B = 4
_N = 512
_L = 16
_NC = 2
_NS = 16
_TPB = (_NC * _NS) // _B
_PPT = _N // _TPB

_OQKV = 0
_OW1 = 27
_OB1 = 123
_OW2 = 155
_OB2 = 251
_OA1 = 254
_OAB1 = 290
_OA2 = 302
_OAB2 = 338
_PER_LAYER = 341
_OLW = 3 * _PER_LAYER
_OLB = _OLW + 6
_NW = _OLB + 2


def _pack_weights(params):
  segs = []
  for ln in ('l1', 'l2', 'l3'):
    p = params[ln]
    segs += [p['qkv'].reshape(-1), p['pos_w1'].reshape(-1), p['pos_b1'],
             p['pos_w2'].reshape(-1), p['pos_b2'], p['attn_w1'].reshape(-1),
             p['attn_b1'], p['attn_w2'].reshape(-1), p['attn_b2']]
  segs += [params['lin_w'].reshape(-1), params['lin_b']]
  w = jnp.concatenate(segs).astype(jnp.float32)
  return jnp.repeat(w, _L)


def _sc_body(pos_hbm, x_hbm, w_hbm, out_hbm,
             posb, xb, qkvb, idxb, rpb, outb, xnb, fb, wv, xsh):
  c = lax.axis_index('c')
  s = lax.axis_index('s')
  bl = s // _TPB
  b = c * 2 + bl
  ch = s % _TPB
  base = ch * _PPT

  pltpu.sync_copy(pos_hbm.at[b], posb)
  pltpu.sync_copy(x_hbm.at[b], xb)
  pltpu.sync_copy(w_hbm, wv)

  def wsp(j):
    return wv[pl.ds(_L * j, _L)]

  iota = lax.iota(jnp.int32, _L)

  def topk_body(i, _):
    gi = base + i
    gidx = jnp.full((_L,), gi, jnp.int32)
    px = plsc.load_gather(posb, [gidx])
    py = plsc.load_gather(posb, [gidx + _N])
    pz = plsc.load_gather(posb, [gidx + 2 * _N])

    def chunk_body(j, carry):
      bk, bv = carry
      off = j * _L
      dx = px - posb[pl.ds(off, _L)]
      dy = py - posb[pl.ds(_N + off, _L)]
      dz = pz - posb[pl.ds(2 * _N + off, _L)]
      d2 = dx * dx + dy * dy + dz * dz
      ck, cv = plsc.sort_key_val(d2, iota + off)
      ckr = lax.rev(ck, (0,))
      cvr = lax.rev(cv, (0,))
      keep = bk <= ckr
      mk = jnp.where(keep, bk, ckr)
      mv = jnp.where(keep, bv, cvr)
      nk, nv = plsc.sort_key_val(mk, mv)
      return (nk, nv)

    bk0 = jnp.full((_L,), jnp.inf, jnp.float32)
    bv0 = jnp.zeros((_L,), jnp.int32)
    _, bv = lax.fori_loop(0, _N // _L, chunk_body, (bk0, bv0))
    idxb[pl.ds(i * _L, _L)] = bv
    gx = plsc.load_gather(posb, [bv])
    gy = plsc.load_gather(posb, [bv + _N])
    gz = plsc.load_gather(posb, [bv + 2 * _N])
    rpb[pl.ds(i * _L, _L)] = px - gx
    rpb[pl.ds((_PPT + i) * _L, _L)] = py - gy
    rpb[pl.ds((2 * _PPT + i) * _L, _L)] = pz - gz
    return 0

  lax.fori_loop(0, _PPT, topk_body, 0)

  for l in range(3):
    lb = l * _PER_LAYER

    def qkv_body(j, _, lb=lb):
      off = j * _L
      x0 = xb[pl.ds(off, _L)]
      x1 = xb[pl.ds(_N + off, _L)]
      x2 = xb[pl.ds(2 * _N + off, _L)]
      for u in range(9):
        qkvb[pl.ds(u * _N + off, _L)] = (x0 * wsp(lb + _OQKV + u) +
                                         x1 * wsp(lb + _OQKV + 9 + u) +
                                         x2 * wsp(lb + _OQKV + 18 + u))
      return 0

    lax.fori_loop(0, _N // _L, qkv_body, 0)

    def pt_body(i, _, lb=lb):
      gi = base + i
      gidx = jnp.full((_L,), gi, jnp.int32)
      nb = idxb[pl.ds(i * _L, _L)]
      qx = plsc.load_gather(qkvb, [gidx])
      qy = plsc.load_gather(qkvb, [gidx + _N])
      qz = plsc.load_gather(qkvb, [gidx + 2 * _N])
      kx = plsc.load_gather(qkvb, [nb + 3 * _N])
      ky = plsc.load_gather(qkvb, [nb + 4 * _N])
      kz = plsc.load_gather(qkvb, [nb + 5 * _N])
      vx = plsc.load_gather(qkvb, [nb + 6 * _N])
      vy = plsc.load_gather(qkvb, [nb + 7 * _N])
      vz = plsc.load_gather(qkvb, [nb + 8 * _N])
      rx = rpb[pl.ds(i * _L, _L)]
      ry = rpb[pl.ds((_PPT + i) * _L, _L)]
      rz = rpb[pl.ds((2 * _PPT + i) * _L, _L)]
      pex = wsp(lb + _OB2 + 0)
      pey = wsp(lb + _OB2 + 1)
      pez = wsp(lb + _OB2 + 2)
      for u in range(_POS_HID):
        h = (rx * wsp(lb + _OW1 + u) + ry * wsp(lb + _OW1 + 32 + u) +
             rz * wsp(lb + _OW1 + 64 + u) + wsp(lb + _OB1 + u))
        h = jnp.maximum(h, 0.0)
        pex = pex + h * wsp(lb + _OW2 + u * 3)
        pey = pey + h * wsp(lb + _OW2 + u * 3 + 1)
        pez = pez + h * wsp(lb + _OW2 + u * 3 + 2)
      sx = qx - kx + pex
      sy = qy - ky + pey
      sz = qz - kz + pez
      ox = wsp(lb + _OAB2 + 0)
      oy = wsp(lb + _OAB2 + 1)
      oz = wsp(lb + _OAB2 + 2)
      for u in range(_ATTN_HID):
        g = (sx * wsp(lb + _OA1 + u) + sy * wsp(lb + _OA1 + 12 + u) +
             sz * wsp(lb + _OA1 + 24 + u) + wsp(lb + _OAB1 + u))
        g = jnp.maximum(g, 0.0)
        ox = ox + g * wsp(lb + _OA2 + u * 3)
        oy = oy + g * wsp(lb + _OA2 + u * 3 + 1)
        oz = oz + g * wsp(lb + _OA2 + u * 3 + 2)

      def chan(sim, vv, pe):
        m = jnp.max(sim)
        e = jnp.exp(sim - m)
        a = e / jnp.sum(e)
        return jnp.sum(a * (vv + pe))

      o0 = chan(ox, vx, pex)
      o1 = chan(oy, vy, pey)
      o2 = chan(oz, vz, pez)
      vout = jnp.where(iota == 0, o0, jnp.where(iota == 1, o1, o2))
      plsc.store_scatter(outb, [jnp.minimum(iota, 2) * _PPT + i],
                         vout, mask=iota < 3)
      return 0

    lax.fori_loop(0, _PPT, pt_body, 0)

    if l < 2:
      for t in range(_PPT // _L):
        for cc in range(3):
          z = outb[pl.ds(cc * _PPT + t * _L, _L)]
          xnb[pl.ds(cc * _PPT + t * _L, _L)] = 1.0 / (1.0 + jnp.exp(-z))
      for cc in range(3):
        pltpu.sync_copy(xnb.at[pl.ds(cc * _PPT, _PPT)],
                        xsh.at[bl, pl.ds(cc * _N + base, _PPT)])
      plsc.subcore_barrier()
      pltpu.sync_copy(xsh.at[bl], xb)
      plsc.subcore_barrier()
    else:
      for t in range(_PPT // _L):
        xs = []
        for cc in range(3):
          z = outb[pl.ds(cc * _PPT + t * _L, _L)]
          xs.append(1.0 / (1.0 + jnp.exp(-z)))
        u0 = (xs[0] * wsp(_OLW + 0) + xs[1] * wsp(_OLW + 2) +
              xs[2] * wsp(_OLW + 4) + wsp(_OLB + 0))
        u1 = (xs[0] * wsp(_OLW + 1) + xs[1] * wsp(_OLW + 3) +
              xs[2] * wsp(_OLW + 5) + wsp(_OLB + 1))
        m = jnp.maximum(u0, u1)
        e0 = jnp.exp(u0 - m)
        e1 = jnp.exp(u1 - m)
        tot = e0 + e1
        fb[pl.ds(t * _L, _L)] = e0 / tot
        fb[pl.ds(_PPT + t * _L, _L)] = e1 / tot
      for o in range(2):
        pltpu.sync_copy(fb.at[pl.ds(o * _PPT, _PPT)],
                        out_hbm.at[b, pl.ds(o * _N + base, _PPT)])


@jax.jit
def _sc_call(pos_t, x_t, wflat):
  mesh = plsc.VectorSubcoreMesh(core_axis_name='c', subcore_axis_name='s',
                                num_cores=_NC, num_subcores=_NS)
  return pl.kernel(
      _sc_body,
      out_type=jax.ShapeDtypeStruct((_B, 2 * _N), jnp.float32),
      mesh=mesh,
      scratch_types=[
          pltpu.VMEM((3 * _N,), jnp.float32),
          pltpu.VMEM((3 * _N,), jnp.float32),
          pltpu.VMEM((9 * _N,), jnp.float32),
          pltpu.VMEM((_PPT * _L,), jnp.int32),
          pltpu.VMEM((3 * _PPT * _L,), jnp.float32),
          pltpu.VMEM((3 * _PPT,), jnp.float32),
          pltpu.VMEM((3 * _PPT,), jnp.float32),
          pltpu.VMEM((2 * _PPT,), jnp.float32),
          pltpu.VMEM((_NW * _L,), jnp.float32),
          pltpu.VMEM_SHARED((2, 3 * _N), jnp.float32),
      ],
      compiler_params=pltpu.CompilerParams(use_tc_tiling_on_sc=False,
                                           needs_layout_passes=False),
      name='pt_knn_sc',
  )(pos_t, x_t, wflat)


def kernel(feats, pos, mask, params):
  del mask
  pos_t = jnp.transpose(pos, (0, 2, 1)).reshape(_B, 3 * _N).astype(jnp.float32)
  x_t = jnp.transpose(feats, (0, 2, 1)).reshape(_B, 3 * _N).astype(jnp.float32)
  wflat = _pack_weights(params)
  out = _sc_call(pos_t, x_t, wflat)
  return jnp.transpose(out.reshape(_B, 2, _N), (0, 2, 1))
